# Initial kernel scaffold; baseline (speedup 1.0000x reference)
import jax
import jax.numpy as jnp
from jax.experimental import pallas as pl


def kernel(node_feats, collated_edge_index, W1, b1, W2, b2, wscore, Wcls, bcls):
    raise NotImplementedError("write your pallas kernel here")


# speedup vs baseline: 6.3821x; 6.3821x over previous
import functools

import jax
import jax.numpy as jnp
from jax import lax
from jax.experimental import pallas as pl
from jax.experimental.pallas import tpu as pltpu
from jax.experimental.pallas import tpu_sc as plsc

B = 50
M = 200
D = 128
N = B * M
E = 320000
K = M // 2
NCLS = 11

NC = 2
NS = 16
HALF = D // NC
ROWS_PER_TILE = N // NS
CHUNK = 128
CHUNKS_PER_TILE = -(-E // (NS * CHUNK))
E_PAD = NS * CHUNKS_PER_TILE * CHUNK
TRASH = N
N_SH = N + 16


def _segsum_plus_x(x, srcg, dstg):
    mesh = plsc.VectorSubcoreMesh(core_axis_name="c", subcore_axis_name="s")

    @functools.partial(
        pl.kernel,
        mesh=mesh,
        out_type=jax.ShapeDtypeStruct((N, D), jnp.float32),
        compiler_params=pltpu.CompilerParams(use_tc_tiling_on_sc=False),
        scratch_types=[
            pltpu.VMEM((CHUNKS_PER_TILE, CHUNK), jnp.int32),
            pltpu.VMEM((CHUNKS_PER_TILE, CHUNK), jnp.int32),
            pltpu.VMEM((CHUNK, HALF), jnp.float32),
            pltpu.VMEM_SHARED((N_SH, HALF), jnp.float32),
            pltpu.VMEM_SHARED((N_SH, HALF), jnp.float32),
        ],
    )
    def seg_kernel(x_hbm, src_hbm, dst_hbm, out_hbm, src_v, dst_v, rows_v,
                   x_sh, agg_sh):
        c = lax.axis_index("c")
        s = lax.axis_index("s")
        col0 = c * HALF
        row0 = s * ROWS_PER_TILE
        pltpu.sync_copy(x_hbm.at[pl.ds(row0, ROWS_PER_TILE), pl.ds(col0, HALF)],
                        x_sh.at[pl.ds(row0, ROWS_PER_TILE), :])
        pltpu.sync_copy(x_hbm.at[pl.ds(row0, ROWS_PER_TILE), pl.ds(col0, HALF)],
                        agg_sh.at[pl.ds(row0, ROWS_PER_TILE), :])
        pltpu.sync_copy(src_hbm.at[s], src_v)
        pltpu.sync_copy(dst_hbm.at[s], dst_v)
        plsc.subcore_barrier()

        def body(j, carry):
            pltpu.sync_copy(x_sh.at[src_v.at[j]], rows_v)
            pltpu.sync_copy(rows_v, agg_sh.at[dst_v.at[j]], add=True)
            return carry

        lax.fori_loop(0, CHUNKS_PER_TILE, body, 0)
        plsc.subcore_barrier()
        pltpu.sync_copy(agg_sh.at[pl.ds(row0, ROWS_PER_TILE), :],
                        out_hbm.at[pl.ds(row0, ROWS_PER_TILE), pl.ds(col0, HALF)])

    return seg_kernel(x, srcg, dstg)


def _dense_relu(h, W, b):
    blk = 2000

    def body(h_ref, w_ref, b_ref, o_ref):
        o_ref[...] = jnp.maximum(
            jnp.dot(h_ref[...], w_ref[...],
                    preferred_element_type=jnp.float32) + b_ref[...], 0.0)

    return pl.pallas_call(
        body,
        grid=(N // blk,),
        in_specs=[pl.BlockSpec((blk, D), lambda i: (i, 0)),
                  pl.BlockSpec((D, D), lambda i: (0, 0)),
                  pl.BlockSpec((1, D), lambda i: (0, 0))],
        out_specs=pl.BlockSpec((blk, D), lambda i: (i, 0)),
        out_shape=jax.ShapeDtypeStruct((N, D), jnp.float32),
    )(h, W, b.reshape(1, D))


def _head(h3, x2, wscore, wcls_pad, bcls_pad):
    def body(h_ref, x_ref, ws_ref, wc_ref, bc_ref, o_ref):
        h = h_ref[...]
        s = jnp.tanh(lax.dot_general(h, ws_ref[...], (((1,), (0,)), ((), ())),
                                     preferred_element_type=jnp.float32))
        ii = lax.broadcasted_iota(jnp.int32, (M, M), 0)
        jj = lax.broadcasted_iota(jnp.int32, (M, M), 1)
        eye = (ii == jj).astype(jnp.float32)
        s_row = lax.dot_general(s, eye, (((0,), (0,)), ((), ())),
                                preferred_element_type=jnp.float32)
        s_col_b = lax.broadcast_in_dim(s, (M, M), (0, 1))
        s_row_b = lax.broadcast_in_dim(s_row, (M, M), (0, 1))
        beats = (s_row_b > s_col_b) | ((s_row_b == s_col_b) & (jj < ii))
        rank = jnp.sum(beats.astype(jnp.float32), axis=1, keepdims=True)
        w = jnp.where(rank < float(K), s, 0.0) * (1.0 / K)
        pooled = lax.dot_general(w, x_ref[...], (((0,), (0,)), ((), ())),
                                 preferred_element_type=jnp.float32)
        emb = jnp.maximum(pooled, 0.0)
        logits = jnp.dot(emb, wc_ref[...],
                         preferred_element_type=jnp.float32) + bc_ref[...]
        o_ref[...] = lax.broadcast_in_dim(logits, (8, D), (0, 1))

    out = pl.pallas_call(
        body,
        grid=(B,),
        in_specs=[pl.BlockSpec((M, D), lambda i: (i, 0)),
                  pl.BlockSpec((M, D), lambda i: (i, 0)),
                  pl.BlockSpec((D, 1), lambda i: (0, 0)),
                  pl.BlockSpec((D, D), lambda i: (0, 0)),
                  pl.BlockSpec((1, D), lambda i: (0, 0))],
        out_specs=pl.BlockSpec((8, D), lambda i: (i, 0)),
        out_shape=jax.ShapeDtypeStruct((B * 8, D), jnp.float32),
    )(h3, x2, wscore.reshape(D, 1), wcls_pad, bcls_pad)
    return out[::8]


def kernel(node_feats, collated_edge_index, W1, b1, W2, b2, wscore, Wcls, bcls):
    x = node_feats.reshape(N, D).astype(jnp.float32)
    src = collated_edge_index[0].astype(jnp.int32)
    dst = collated_edge_index[1].astype(jnp.int32)
    pad = E_PAD - E
    srcg = jnp.concatenate([src, jnp.zeros((pad,), jnp.int32)]).reshape(
        NS, CHUNKS_PER_TILE, CHUNK)
    dstg = jnp.concatenate([dst, jnp.full((pad,), TRASH, jnp.int32)]).reshape(
        NS, CHUNKS_PER_TILE, CHUNK)

    h1 = _segsum_plus_x(x, srcg, dstg)
    x1 = _dense_relu(h1, W1, b1)
    h2 = _segsum_plus_x(x1, srcg, dstg)
    x2 = _dense_relu(h2, W2, b2)
    h3 = _segsum_plus_x(x2, srcg, dstg)

    wcls_pad = jnp.zeros((D, D), jnp.float32).at[:, :NCLS].set(Wcls)
    bcls_pad = jnp.zeros((1, D), jnp.float32).at[0, :NCLS].set(bcls)
    out = _head(h3, x2, wscore, wcls_pad, bcls_pad)
    return out[:, :NCLS]
